# baseline (speedup 1.0000x reference)
import functools

import jax
import jax.numpy as jnp
from jax import lax
from jax.experimental import pallas as pl
from jax.experimental.pallas import tpu as pltpu
from jax.experimental.pallas import tpu_sc as plsc

_INFO = plsc.get_sparse_core_info()
_NC = _INFO.num_cores
_NS = _INFO.num_subcores
_NW = _NC * _NS
_L = _INFO.num_lanes
_WIN = 128
_D = 8


def _make_sc_call(B, EMB, V):
    b_per_w = B // _NW
    n_groups = b_per_w // _L
    tr = EMB // 8
    mesh = plsc.VectorSubcoreMesh(core_axis_name="c", subcore_axis_name="s")

    @functools.partial(
        pl.kernel,
        out_type=jax.ShapeDtypeStruct((B,), jnp.float32),
        mesh=mesh,
        compiler_params=pltpu.CompilerParams(needs_layout_passes=False),
        scratch_types=[
            pltpu.VMEM((b_per_w,), jnp.int32),
            pltpu.VMEM((b_per_w,), jnp.int32),
            pltpu.VMEM((_D, tr, 8, _WIN), jnp.float32),
            pltpu.VMEM((_D, tr, 8, _WIN), jnp.float32),
            pltpu.VMEM((b_per_w,), jnp.float32),
            pltpu.VMEM((_L,), jnp.float32),
            pltpu.VMEM((_L,), jnp.float32),
        ] + [pltpu.SemaphoreType.DMA] * _D,
    )
    def sc_call(xr, user_t, movie_t, wb, out,
                idx_u, idx_m, win_u, win_m, out_v, w_v, b_v,
                *sems):
        wid = lax.axis_index("s") * _NC + lax.axis_index("c")
        base = wid * b_per_w

        pltpu.sync_copy(xr.at[0, wid], idx_u)
        pltpu.sync_copy(xr.at[1, wid], idx_m)
        pltpu.sync_copy(wb.at[0], w_v)
        pltpu.sync_copy(wb.at[1], b_v)

        def fire(iu, im, parity):
            sem = sems[parity]
            ou = pl.multiple_of(iu - lax.rem(iu, _WIN), _WIN)
            om = pl.multiple_of(im - lax.rem(im, _WIN), _WIN)
            pltpu.async_copy(
                user_t.at[:, :, pl.ds(ou, _WIN)], win_u.at[parity], sem)
            pltpu.async_copy(
                movie_t.at[:, :, pl.ds(om, _WIN)], win_m.at[parity], sem)

        def wait(parity):
            sem = sems[parity]
            pltpu.make_async_copy(
                user_t.at[:, :, pl.ds(0, _WIN)], win_u.at[parity], sem).wait()
            pltpu.make_async_copy(
                movie_t.at[:, :, pl.ds(0, _WIN)], win_m.at[parity], sem).wait()

        w = w_v[...]
        b = b_v[...]
        lanes = lax.iota(jnp.int32, _L)
        tc_lo = lanes // 8
        tc_hi = tc_lo + 2
        ec = lax.rem(lanes, 8)

        def pick(v, lane):
            return jnp.sum(jnp.where(lanes == lane, v, 0))

        iuv0 = idx_u[pl.ds(0, _L)]
        imv0 = idx_m[pl.ds(0, _L)]
        for k in range(_D - 1):
            fire(pick(iuv0, k), pick(imv0, k), k % _D)

        def group(g, carry):
            i0 = g * _L
            iuv = idx_u[pl.ds(i0, _L)]
            imv = idx_m[pl.ds(i0, _L)]
            i1 = jnp.minimum(i0 + _L, b_per_w - _L)
            iuv_n = idx_u[pl.ds(i1, _L)]
            imv_n = idx_m[pl.ds(i1, _L)]
            res = jnp.zeros((_L,), jnp.float32)
            cur_u = [pick(iuv, k) for k in range(_D - 1)]
            cur_m = [pick(imv, k) for k in range(_D - 1)]
            for j in range(_L):
                p = j % _D
                ja = j + _D - 1
                if ja < _L:
                    nxt_iu = pick(iuv, ja)
                    nxt_im = pick(imv, ja)
                else:
                    nxt_iu = pick(iuv_n, ja - _L)
                    nxt_im = pick(imv_n, ja - _L)
                fire(nxt_iu, nxt_im, ja % _D)
                wait(p)
                cu = jnp.full((_L,), lax.rem(cur_u[0], _WIN), jnp.int32)
                cm = jnp.full((_L,), lax.rem(cur_m[0], _WIN), jnp.int32)
                u_lo = plsc.load_gather(win_u.at[p], [tc_lo, ec, cu])
                u_hi = plsc.load_gather(win_u.at[p], [tc_hi, ec, cu])
                m_lo = plsc.load_gather(win_m.at[p], [tc_lo, ec, cm])
                m_hi = plsc.load_gather(win_m.at[p], [tc_hi, ec, cm])
                prod = u_lo * m_lo + u_hi * m_hi
                s = jnp.sum(prod)
                res = jnp.where(lanes == j, s, res)
                cur_u = cur_u[1:] + [nxt_iu]
                cur_m = cur_m[1:] + [nxt_im]
            z = res * w + b
            out_v[pl.ds(i0, _L)] = 1.0 / (1.0 + jnp.exp(-z))
            return carry

        lax.fori_loop(0, n_groups, group, 0)
        for k in range(_D - 1):
            wait((b_per_w + k) % _D)

        pltpu.sync_copy(out_v, out.at[pl.ds(base, b_per_w)])

    return sc_call


def kernel(x, user_table, movie_table, W_fc, b_fc):
    B = x.shape[1]
    V, EMB = user_table.shape
    xr = x.astype(jnp.int32).reshape(2, _NW, B // _NW)
    ut = user_table.T.reshape(EMB // 8, 8, V)
    mt = movie_table.T.reshape(EMB // 8, 8, V)
    wb = jnp.stack([
        jnp.broadcast_to(W_fc.reshape(()), (_L,)),
        jnp.broadcast_to(b_fc.reshape(()), (_L,)),
    ]).astype(jnp.float32)
    out = _make_sc_call(B, EMB, V)(xr, ut, mt, wb)
    return out.reshape(B, 1)
